# speedup vs baseline: 7.1182x; 7.1182x over previous
import functools

import jax
import jax.numpy as jnp
from jax import lax
from jax.experimental import pallas as pl
from jax.experimental.pallas import tpu as pltpu
from jax.experimental.pallas import tpu_sc as plsc

_N = 10000
_D = 128
_E = 320000
_B = 2 * _E
_NW = 32
_B_PER_W = _B // _NW
_CHUNK = 1000
_NSTEPS = _B_PER_W // _CHUNK

_mesh = plsc.VectorSubcoreMesh(core_axis_name="c", subcore_axis_name="s")


@functools.partial(
    pl.kernel,
    out_type=jax.ShapeDtypeStruct((_B, _D), jnp.float32),
    mesh=_mesh,
    scratch_types=[
        pltpu.VMEM((_CHUNK,), jnp.int32),
        pltpu.VMEM((_CHUNK, _D), jnp.float32),
        pltpu.SemaphoreType.DMA,
    ],
)
def _gather_rows(table_hbm, idx_hbm, out_hbm, idx_v, rows_v, sem):
    wid = lax.axis_index("s") * 2 + lax.axis_index("c")
    base = wid * _B_PER_W

    @pl.loop(0, _NSTEPS)
    def _step(i):
        off = base + i * _CHUNK
        pltpu.sync_copy(idx_hbm.at[pl.ds(off, _CHUNK)], idx_v)
        pltpu.async_copy(table_hbm.at[idx_v], rows_v, sem).wait()
        pltpu.sync_copy(rows_v, out_hbm.at[pl.ds(off, _CHUNK)])


@jax.jit
def kernel(vertices, edges, edge_features, edge_matrices):
    del edge_features, edge_matrices
    idx = edges.reshape(_B)
    out = _gather_rows(vertices, idx)
    return out.reshape(2, _E, _D)


# speedup vs baseline: 7.3783x; 1.0365x over previous
import functools

import jax
import jax.numpy as jnp
from jax import lax
from jax.experimental import pallas as pl
from jax.experimental.pallas import tpu as pltpu
from jax.experimental.pallas import tpu_sc as plsc

_N = 10000
_D = 128
_E = 320000
_B = 2 * _E
_NW = 32
_B_PER_W = _B // _NW
_CHUNK = 400
_NSTEPS = _B_PER_W // _CHUNK
_NBUF = 2

_mesh = plsc.VectorSubcoreMesh(core_axis_name="c", subcore_axis_name="s")


@functools.partial(
    pl.kernel,
    out_type=jax.ShapeDtypeStruct((_B, _D), jnp.float32),
    mesh=_mesh,
    scratch_types=[
        [pltpu.VMEM((_CHUNK,), jnp.int32)] * _NBUF,
        [pltpu.VMEM((_CHUNK, _D), jnp.float32)] * _NBUF,
        [pltpu.SemaphoreType.DMA] * _NBUF,
    ],
)
def _gather_rows(table_hbm, idx_hbm, out_hbm, idx_v, rows_v, sems):
    wid = lax.axis_index("s") * 2 + lax.axis_index("c")
    base = wid * _B_PER_W

    def start_gather(step, b):
        off = base + step * _CHUNK
        pltpu.sync_copy(idx_hbm.at[pl.ds(off, _CHUNK)], idx_v[b])
        pltpu.async_copy(table_hbm.at[idx_v[b]], rows_v[b], sems[b])

    def wait_gather(b):
        pltpu.make_async_copy(table_hbm.at[idx_v[b]], rows_v[b],
                              sems[b]).wait()

    def store(step, b):
        off = base + step * _CHUNK
        pltpu.sync_copy(rows_v[b], out_hbm.at[pl.ds(off, _CHUNK)])

    for b in range(_NBUF):
        start_gather(b, b)

    @pl.loop(0, _NSTEPS - _NBUF, step=_NBUF)
    def _steps(i):
        for b in range(_NBUF):
            step = i + b
            wait_gather(b)
            store(step, b)
            start_gather(step + _NBUF, b)

    for b in range(_NBUF):
        step = _NSTEPS - _NBUF + b
        wait_gather(b)
        store(step, b)


@jax.jit
def kernel(vertices, edges, edge_features, edge_matrices):
    del edge_features, edge_matrices
    idx = edges.reshape(_B)
    out = _gather_rows(vertices, idx)
    return out.reshape(2, _E, _D)
